# baseline (speedup 1.0000x reference)
import functools

import jax
import jax.numpy as jnp
from jax import lax
from jax.experimental import pallas as pl
from jax.experimental.pallas import tpu as pltpu
from jax.experimental.pallas import tpu_sc as plsc


def _sc_gather_sum(input_text, emb):
    B, L = input_text.shape
    V, D = emb.shape
    info = plsc.get_sparse_core_info()
    NC, NS = info.num_cores, info.num_subcores
    NW = NC * NS
    assert B % NW == 0
    TPB = B // NW
    CH0 = min(128, L)
    CH1 = L - CH0
    assert CH0 % 8 == 0 and (CH1 == 0 or CH1 % 8 == 0)
    NV = D // 16

    mesh = plsc.VectorSubcoreMesh(core_axis_name="c", subcore_axis_name="s")

    @functools.partial(
        pl.kernel,
        out_type=jax.ShapeDtypeStruct((B, D), jnp.float32),
        mesh=mesh,
        compiler_params=pltpu.CompilerParams(use_tc_tiling_on_sc=False),
        scratch_types=[
            pltpu.VMEM((TPB, L), jnp.int32),
            pltpu.VMEM((L, D), jnp.float32),
            pltpu.VMEM((TPB, D), jnp.float32),
            pltpu.SemaphoreType.DMA,
        ],
    )
    def sc_sum(text_hbm, emb_hbm, out_hbm, idx_v, rows_v, out_v, sem):
        wid = lax.axis_index("s") * NC + lax.axis_index("c")
        base = pl.multiple_of(wid * TPB, 8)
        pltpu.sync_copy(text_hbm.at[pl.ds(base, TPB)], idx_v)

        def text_body(t, carry):
            c0 = pltpu.async_copy(
                emb_hbm.at[idx_v.at[t, pl.ds(0, CH0)]],
                rows_v.at[pl.ds(0, CH0)], sem)
            if CH1:
                c1 = pltpu.async_copy(
                    emb_hbm.at[idx_v.at[t, pl.ds(CH0, CH1)]],
                    rows_v.at[pl.ds(CH0, CH1)], sem)
            c0.wait()
            if CH1:
                c1.wait()

            def row_body(j, acc):
                return tuple(acc[p] + rows_v[j, pl.ds(p * 16, 16)]
                             for p in range(NV))

            acc = lax.fori_loop(
                0, L, row_body,
                tuple(jnp.zeros((16,), jnp.float32) for _ in range(NV)),
                unroll=8)
            for p in range(NV):
                out_v[t, pl.ds(p * 16, 16)] = acc[p]
            return carry

        lax.fori_loop(0, TPB, text_body, 0)
        pltpu.sync_copy(out_v, out_hbm.at[pl.ds(base, TPB)])

    return sc_sum(input_text, emb)


def _mlp_body(sum_ref, len_ref, w1_ref, b1_ref, w2_ref, b2_ref, out_ref):
    avg = sum_ref[...] / len_ref[...]
    h = lax.dot_general(avg, w1_ref[...], (((1,), (1,)), ((), ())),
                        preferred_element_type=jnp.float32) + b1_ref[...]
    h = jnp.maximum(h, 0.0)
    out_ref[...] = lax.dot_general(h, w2_ref[...], (((1,), (1,)), ((), ())),
                                   preferred_element_type=jnp.float32) + b2_ref[...]


def _mlp(summed, lenf, W1, b1, W2, b2):
    B, D = summed.shape
    H = W1.shape[0]
    C = W2.shape[0]
    BT = 512
    grid = (B // BT,)
    return pl.pallas_call(
        _mlp_body,
        grid=grid,
        in_specs=[
            pl.BlockSpec((BT, D), lambda i: (i, 0)),
            pl.BlockSpec((BT, 1), lambda i: (i, 0)),
            pl.BlockSpec((H, D), lambda i: (0, 0)),
            pl.BlockSpec((1, H), lambda i: (0, 0)),
            pl.BlockSpec((C, H), lambda i: (0, 0)),
            pl.BlockSpec((1, C), lambda i: (0, 0)),
        ],
        out_specs=pl.BlockSpec((BT, C), lambda i: (i, 0)),
        out_shape=jax.ShapeDtypeStruct((B, C), jnp.float32),
    )(summed, lenf, W1, b1.reshape(1, H), W2, b2.reshape(1, C))


def kernel(input_text, text_len, emb, W1, b1, W2, b2):
    B = input_text.shape[0]
    summed = _sc_gather_sum(input_text, emb)
    lenf = text_len.astype(jnp.float32).reshape(B, 1)
    return _mlp(summed, lenf, W1, b1, W2, b2)
